# baseline (speedup 1.0000x reference)
import functools

import jax
import jax.numpy as jnp
from jax import lax
from jax.experimental import pallas as pl
from jax.experimental.pallas import tpu as pltpu
from jax.experimental.pallas import tpu_sc as plsc

_NUM_CORES = 2
_NUM_SUBCORES = 16
_NUM_WORKERS = _NUM_CORES * _NUM_SUBCORES


@functools.cache
def _make_gather(n_rows: int, vocab: int, dim: int):
  assert n_rows % _NUM_WORKERS == 0
  rows_per_w = n_rows // _NUM_WORKERS
  chunk = 1600
  while rows_per_w % chunk:
    chunk //= 2
  n_chunks = rows_per_w // chunk

  mesh = plsc.VectorSubcoreMesh(
      core_axis_name="c", subcore_axis_name="s", num_cores=_NUM_CORES
  )

  @functools.partial(
      pl.kernel,
      mesh=mesh,
      out_type=jax.ShapeDtypeStruct((n_rows, dim), jnp.float32),
      scratch_types=[
          pltpu.VMEM((rows_per_w,), jnp.int32),
          pltpu.VMEM((chunk, dim), jnp.float32),
          pltpu.VMEM((chunk, dim), jnp.float32),
          pltpu.SemaphoreType.DMA,
          pltpu.SemaphoreType.DMA,
          pltpu.SemaphoreType.DMA,
          pltpu.SemaphoreType.DMA,
      ],
      compiler_params=pltpu.CompilerParams(use_tc_tiling_on_sc=False),
  )
  def gather(table_hbm, idx_hbm, out_hbm, idx_v, rows0, rows1, sg0, sg1,
             so0, so1):
    wid = lax.axis_index("s") * _NUM_CORES + lax.axis_index("c")
    base = wid * rows_per_w
    rows = (rows0, rows1)
    sem_g = (sg0, sg1)
    sem_o = (so0, so1)

    pltpu.sync_copy(idx_hbm.at[pl.ds(base, rows_per_w)], idx_v)

    g = [None, None]
    o = [None, None]
    g[0] = pltpu.async_copy(
        table_hbm.at[idx_v.at[pl.ds(0, chunk)]], rows0, sg0)
    for i in range(1, n_chunks + 1):
      b, pb = i % 2, (i - 1) % 2
      g[pb].wait()
      o_new = pltpu.async_copy(
          rows[pb], out_hbm.at[pl.ds(base + (i - 1) * chunk, chunk)],
          sem_o[pb])
      if i < n_chunks:
        if o[b] is not None:
          o[b].wait()
        g[b] = pltpu.async_copy(
            table_hbm.at[idx_v.at[pl.ds(i * chunk, chunk)]], rows[b],
            sem_g[b])
      o[pb] = o_new
    o[0].wait()
    o[1].wait()

  return gather


def kernel(x, table):
  b, h = x.shape
  vocab, dim = table.shape
  idx = x.reshape(-1).astype(jnp.int32)
  out = _make_gather(b * h, vocab, dim)(table, idx)
  return out.reshape(b, h, dim)


# speedup vs baseline: 1.0038x; 1.0038x over previous
import functools

import jax
import jax.numpy as jnp
from jax import lax
from jax.experimental import pallas as pl
from jax.experimental.pallas import tpu as pltpu
from jax.experimental.pallas import tpu_sc as plsc

_NUM_CORES = 2
_NUM_SUBCORES = 16
_NUM_WORKERS = _NUM_CORES * _NUM_SUBCORES


@functools.cache
def _make_gather(n_rows: int, vocab: int, dim: int):
  assert n_rows % _NUM_WORKERS == 0
  rows_per_w = n_rows // _NUM_WORKERS
  nbuf = 4
  chunk = 800
  while rows_per_w % chunk:
    chunk //= 2
  n_chunks = rows_per_w // chunk
  assert n_chunks >= nbuf

  mesh = plsc.VectorSubcoreMesh(
      core_axis_name="c", subcore_axis_name="s", num_cores=_NUM_CORES
  )

  @functools.partial(
      pl.kernel,
      mesh=mesh,
      out_type=jax.ShapeDtypeStruct((n_rows, dim), jnp.float32),
      scratch_types=[
          pltpu.VMEM((rows_per_w,), jnp.int32),
          [pltpu.VMEM((chunk, dim), jnp.float32)] * nbuf,
          [pltpu.SemaphoreType.DMA] * nbuf,
          [pltpu.SemaphoreType.DMA] * nbuf,
      ],
      compiler_params=pltpu.CompilerParams(use_tc_tiling_on_sc=False),
  )
  def gather(table_hbm, idx_hbm, out_hbm, idx_v, rows, sem_g, sem_o):
    wid = lax.axis_index("s") * _NUM_CORES + lax.axis_index("c")
    base = wid * rows_per_w

    pltpu.sync_copy(idx_hbm.at[pl.ds(base, rows_per_w)], idx_v)

    def start_gather(i, b):
      return pltpu.async_copy(
          table_hbm.at[idx_v.at[pl.ds(i * chunk, chunk)]], rows[b],
          sem_g[b])

    g = [start_gather(i, i) for i in range(nbuf)]
    o = [None] * nbuf
    for i in range(n_chunks):
      b = i % nbuf
      g[b].wait()
      o[b] = pltpu.async_copy(
          rows[b], out_hbm.at[pl.ds(base + i * chunk, chunk)], sem_o[b])
      nxt = i + nbuf
      if nxt < n_chunks:
        o[b].wait()
        o[b] = None
        g[b] = start_gather(nxt, b)
    for b in range(nbuf):
      if o[b] is not None:
        o[b].wait()

  return gather


def kernel(x, table):
  b, h = x.shape
  vocab, dim = table.shape
  idx = x.reshape(-1).astype(jnp.int32)
  out = _make_gather(b * h, vocab, dim)(table, idx)
  return out.reshape(b, h, dim)


# speedup vs baseline: 1.6113x; 1.6052x over previous
import functools

import jax
import jax.numpy as jnp
from jax import lax
from jax.experimental import pallas as pl
from jax.experimental.pallas import tpu as pltpu
from jax.experimental.pallas import tpu_sc as plsc

_NUM_CORES = 2
_NUM_SUBCORES = 16
_NUM_WORKERS = _NUM_CORES * _NUM_SUBCORES
_LANES = 16


@functools.cache
def _make_lookup(batch: int, hist: int, vocab: int, dim: int):
  assert batch % _NUM_WORKERS == 0
  b_per_w = batch // _NUM_WORKERS
  cb = 16
  while b_per_w % cb:
    cb //= 2
  n_chunks = b_per_w // cb
  nbuf = min(2, n_chunks)

  mesh = plsc.VectorSubcoreMesh(
      core_axis_name="c", subcore_axis_name="s", num_cores=_NUM_CORES
  )

  @functools.partial(
      pl.kernel,
      mesh=mesh,
      out_type=jax.ShapeDtypeStruct((batch, hist, dim), jnp.float32),
      scratch_types=[
          pltpu.VMEM((b_per_w, hist), jnp.int32),
          pltpu.VMEM((b_per_w * hist,), jnp.int32),
          [pltpu.VMEM((cb * hist, dim), jnp.float32)] * nbuf,
          [pltpu.SemaphoreType.DMA] * nbuf,
          [pltpu.SemaphoreType.DMA] * nbuf,
      ],
      compiler_params=pltpu.CompilerParams(use_tc_tiling_on_sc=False),
  )
  def lookup(x_hbm, table_hbm, out_hbm, x_v, idx_v, rows, sem_g, sem_o):
    wid = lax.axis_index("s") * _NUM_CORES + lax.axis_index("c")
    base = wid * b_per_w

    pltpu.sync_copy(x_hbm.at[pl.ds(base, b_per_w)], x_v)

    n_full = hist // _LANES
    tail = hist - n_full * _LANES

    def compact_row(j, _):
      for k in range(n_full):
        idx_v[pl.ds(j * hist + k * _LANES, _LANES)] = (
            x_v[j, pl.ds(k * _LANES, _LANES)])
      if tail:
        off = hist - _LANES
        idx_v[pl.ds(j * hist + off, _LANES)] = x_v[j, pl.ds(off, _LANES)]
      return 0

    lax.fori_loop(0, b_per_w, compact_row, 0, unroll=4)

    def start_gather(i, b):
      return pltpu.async_copy(
          table_hbm.at[idx_v.at[pl.ds(i * cb * hist, cb * hist)]],
          rows[b], sem_g[b])

    g = [start_gather(i, i) for i in range(nbuf)]
    o = [None] * nbuf
    for i in range(n_chunks):
      b = i % nbuf
      g[b].wait()
      outs = []
      for j in range(cb):
        outs.append(pltpu.async_copy(
            rows[b].at[pl.ds(j * hist, hist), :],
            out_hbm.at[base + i * cb + j], sem_o[b]))
      o[b] = outs
      nxt = i + nbuf
      if nxt < n_chunks:
        for d in o[b]:
          d.wait()
        o[b] = None
        g[b] = start_gather(nxt, b)
    for b in range(nbuf):
      if o[b] is not None:
        for d in o[b]:
          d.wait()

  return lookup


def kernel(x, table):
  b, h = x.shape
  vocab, dim = table.shape
  return _make_lookup(b, h, vocab, dim)(x.astype(jnp.int32), table)


# speedup vs baseline: 1.6113x; 1.0000x over previous
import functools

import jax
import jax.numpy as jnp
from jax import lax
from jax.experimental import pallas as pl
from jax.experimental.pallas import tpu as pltpu
from jax.experimental.pallas import tpu_sc as plsc

_NUM_CORES = 2
_NUM_SUBCORES = 16
_NUM_WORKERS = _NUM_CORES * _NUM_SUBCORES
_LANES = 16
_XPAD = 128


@functools.cache
def _make_lookup(batch: int, hist: int, vocab: int, dim: int):
  assert batch % _NUM_WORKERS == 0
  b_per_w = batch // _NUM_WORKERS
  cb = 16
  while b_per_w % cb:
    cb //= 2
  n_chunks = b_per_w // cb
  nbuf = min(2, n_chunks)
  stage = b_per_w
  while stage * _XPAD * 4 > 128 * 1024:
    stage //= 2
  n_stage = b_per_w // stage

  mesh = plsc.VectorSubcoreMesh(
      core_axis_name="c", subcore_axis_name="s", num_cores=_NUM_CORES
  )

  @functools.partial(
      pl.kernel,
      mesh=mesh,
      out_type=jax.ShapeDtypeStruct((batch, hist, dim), jnp.float32),
      scratch_types=[
          pltpu.VMEM((stage, _XPAD), jnp.int32),
          pltpu.VMEM((b_per_w * hist,), jnp.int32),
          [pltpu.VMEM((cb * hist, dim), jnp.float32)] * nbuf,
          [pltpu.SemaphoreType.DMA] * nbuf,
          [pltpu.SemaphoreType.DMA] * nbuf,
      ],
      compiler_params=pltpu.CompilerParams(use_tc_tiling_on_sc=False),
  )
  def lookup(x_hbm, table_hbm, out_hbm, x_v, idx_v, rows, sem_g, sem_o):
    wid = lax.axis_index("s") * _NUM_CORES + lax.axis_index("c")
    base = wid * b_per_w

    n_full = hist // _LANES
    tail_off = hist - _LANES if hist % _LANES else None

    for h in range(n_stage):
      pltpu.sync_copy(x_hbm.at[pl.ds(base + h * stage, stage)], x_v)
      h_base = h * stage * hist

      def compact_row(j, _):
        dst = h_base + j * hist
        for k in range(n_full):
          idx_v[pl.ds(dst + k * _LANES, _LANES)] = (
              x_v[j, pl.ds(k * _LANES, _LANES)])
        if tail_off is not None:
          idx_v[pl.ds(dst + tail_off, _LANES)] = (
              x_v[j, pl.ds(tail_off, _LANES)])
        return 0

      lax.fori_loop(0, stage, compact_row, 0, unroll=4)

    def start_gather(i, b):
      return pltpu.async_copy(
          table_hbm.at[idx_v.at[pl.ds(i * cb * hist, cb * hist)]],
          rows[b], sem_g[b])

    g = [start_gather(i, i) for i in range(nbuf)]
    o = [None] * nbuf
    for i in range(n_chunks):
      b = i % nbuf
      g[b].wait()
      outs = []
      for j in range(cb):
        outs.append(pltpu.async_copy(
            rows[b].at[pl.ds(j * hist, hist), :],
            out_hbm.at[base + i * cb + j], sem_o[b]))
      o[b] = outs
      nxt = i + nbuf
      if nxt < n_chunks:
        for d in o[b]:
          d.wait()
        o[b] = None
        g[b] = start_gather(nxt, b)
    for b in range(nbuf):
      if o[b] is not None:
        for d in o[b]:
          d.wait()

  return lookup


def kernel(x, table):
  b, h = x.shape
  vocab, dim = table.shape
  xp = jnp.pad(x.astype(jnp.int32), ((0, 0), (0, _XPAD - h)))
  return _make_lookup(b, h, vocab, dim)(xp, table)
